# initial kernel scaffold (unmeasured)
import jax
import jax.numpy as jnp
from jax import lax
from jax.experimental import pallas as pl
from jax.experimental.pallas import tpu as pltpu

N_DEV = 32


def kernel(x, w_mat, scale_x, scale_w):
    m, k_per = x.shape
    k_blk, n = w_mat.shape
    m_blk = m // N_DEV

    comm_dtype = jnp.float8_e5m2

    def body(x_ref, w_ref, sx_ref, sw_ref, out_ref,
             xbuf, wbuf, send_x, recv_x, send_w, recv_w):
        my = lax.axis_index("i")
        left = (my + N_DEV - 1) % N_DEV
        right = (my + 1) % N_DEV

        barrier = pltpu.get_barrier_semaphore()
        for nbr in (left, right):
            pl.semaphore_signal(barrier, inc=1, device_id=(nbr,),
                                device_id_type=pl.DeviceIdType.MESH)
        pl.semaphore_wait(barrier, 2)

        xbuf[0, :, :] = x_ref[...].astype(comm_dtype)
        wbuf[0, :, :] = w_ref[...].astype(comm_dtype)

        row0 = my * m_blk

        def accum(slot, first):
            tile = xbuf[slot, pl.ds(row0, m_blk), :].astype(jnp.bfloat16)
            part = jnp.dot(tile, wbuf[slot].astype(jnp.bfloat16),
                           preferred_element_type=jnp.float32)
            if first:
                out_ref[...] = part
            else:
                out_ref[...] += part

        accum(0, first=True)

        for h in range(N_DEV - 1):
            rx = pltpu.make_async_remote_copy(
                src_ref=xbuf.at[h], dst_ref=xbuf.at[h + 1],
                send_sem=send_x.at[h], recv_sem=recv_x.at[h + 1],
                device_id=(right,), device_id_type=pl.DeviceIdType.MESH)
            rw = pltpu.make_async_remote_copy(
                src_ref=wbuf.at[h], dst_ref=wbuf.at[h + 1],
                send_sem=send_w.at[h], recv_sem=recv_w.at[h + 1],
                device_id=(right,), device_id_type=pl.DeviceIdType.MESH)
            rx.start()
            rw.start()
            rx.wait()
            rw.wait()
            accum(h + 1, first=False)

        s = sx_ref[0] * sw_ref[0]
        y = out_ref[...] * s
        z = jnp.clip(y, -60.0, 60.0)
        out_ref[...] = y / (1.0 + jnp.exp(-z))

    return pl.pallas_call(
        body,
        out_shape=jax.ShapeDtypeStruct((m_blk, n), jnp.float32),
        in_specs=[
            pl.BlockSpec(memory_space=pltpu.VMEM),
            pl.BlockSpec(memory_space=pltpu.VMEM),
            pl.BlockSpec(memory_space=pltpu.SMEM),
            pl.BlockSpec(memory_space=pltpu.SMEM),
        ],
        out_specs=pl.BlockSpec(memory_space=pltpu.VMEM),
        scratch_shapes=[
            pltpu.VMEM((N_DEV, m, k_per), comm_dtype),
            pltpu.VMEM((N_DEV, k_blk, n), comm_dtype),
            pltpu.SemaphoreType.DMA((N_DEV,)),
            pltpu.SemaphoreType.DMA((N_DEV,)),
            pltpu.SemaphoreType.DMA((N_DEV,)),
            pltpu.SemaphoreType.DMA((N_DEV,)),
        ],
        compiler_params=pltpu.CompilerParams(collective_id=0),
    )(x, w_mat, scale_x, scale_w)


# baseline (device time: 746859 ns/iter reference)
import jax
import jax.numpy as jnp
from jax import lax
from jax.experimental import pallas as pl
from jax.experimental.pallas import tpu as pltpu

N_DEV = 32


def kernel(x, w_mat, scale_x, scale_w):
    m, k_per = x.shape
    k_blk, n = w_mat.shape
    m_blk = m // N_DEV

    comm_dtype = jnp.float8_e5m2

    def body(x_ref, w_ref, sx_ref, sw_ref, out_ref,
             xbuf, wbuf, send_x, recv_x, send_w, recv_w, credit):
        my = lax.axis_index("i")
        left = (my + N_DEV - 1) % N_DEV
        right = (my + 1) % N_DEV

        barrier = pltpu.get_barrier_semaphore()
        for nbr in (left, right):
            pl.semaphore_signal(barrier, inc=1, device_id=(nbr,),
                                device_id_type=pl.DeviceIdType.MESH)
        pl.semaphore_wait(barrier, 2)

        xbuf[0, :, :] = x_ref[...].astype(comm_dtype)
        wbuf[0, :, :] = w_ref[...].astype(comm_dtype)

        row0 = my * m_blk

        def accum(slot, first):
            tile = xbuf[slot, pl.ds(row0, m_blk), :].astype(jnp.bfloat16)
            part = jnp.dot(tile, wbuf[slot].astype(jnp.bfloat16),
                           preferred_element_type=jnp.float32)
            if first:
                out_ref[...] = part
            else:
                out_ref[...] += part

        accum(0, first=True)

        for h in range(N_DEV - 1):
            s = h % 2
            r = (h + 1) % 2
            if h >= 2:
                pl.semaphore_wait(credit, 1)
            rx = pltpu.make_async_remote_copy(
                src_ref=xbuf.at[s], dst_ref=xbuf.at[r],
                send_sem=send_x.at[s], recv_sem=recv_x.at[r],
                device_id=(right,), device_id_type=pl.DeviceIdType.MESH)
            rw = pltpu.make_async_remote_copy(
                src_ref=wbuf.at[s], dst_ref=wbuf.at[r],
                send_sem=send_w.at[s], recv_sem=recv_w.at[r],
                device_id=(right,), device_id_type=pl.DeviceIdType.MESH)
            rx.start()
            rw.start()
            rx.wait()
            rw.wait()
            if 1 <= h <= 29:
                pl.semaphore_signal(credit, inc=1, device_id=(left,),
                                    device_id_type=pl.DeviceIdType.MESH)
            accum(r, first=False)

        s = sx_ref[0] * sw_ref[0]
        y = out_ref[...] * s
        z = jnp.clip(y, -60.0, 60.0)
        out_ref[...] = y / (1.0 + jnp.exp(-z))

    return pl.pallas_call(
        body,
        out_shape=jax.ShapeDtypeStruct((m_blk, n), jnp.float32),
        in_specs=[
            pl.BlockSpec(memory_space=pltpu.VMEM),
            pl.BlockSpec(memory_space=pltpu.VMEM),
            pl.BlockSpec(memory_space=pltpu.SMEM),
            pl.BlockSpec(memory_space=pltpu.SMEM),
        ],
        out_specs=pl.BlockSpec(memory_space=pltpu.VMEM),
        scratch_shapes=[
            pltpu.VMEM((2, m, k_per), comm_dtype),
            pltpu.VMEM((2, k_blk, n), comm_dtype),
            pltpu.SemaphoreType.DMA((2,)),
            pltpu.SemaphoreType.DMA((2,)),
            pltpu.SemaphoreType.DMA((2,)),
            pltpu.SemaphoreType.DMA((2,)),
            pltpu.SemaphoreType.REGULAR,
        ],
        compiler_params=pltpu.CompilerParams(collective_id=0),
    )(x, w_mat, scale_x, scale_w)


# device time: 592165 ns/iter; 1.2612x vs baseline; 1.2612x over previous
import jax
import jax.numpy as jnp
from jax import lax
from jax.experimental import pallas as pl
from jax.experimental.pallas import tpu as pltpu

N_DEV = 32
H_L = N_DEV // 2
H_R = N_DEV - 1 - H_L
OWN = 2


def kernel(x, w_mat, scale_x, scale_w):
    m, k_per = x.shape
    k_blk, n = w_mat.shape
    m_blk = m // N_DEV

    comm_dtype = jnp.float8_e5m2

    def body(x_ref, w_ref, sx_ref, sw_ref, out_ref,
             xbl, wbl, xbr, wbr,
             sxl, rxl, swl, rwl, sxr, rxr, swr, rwr,
             credit_l, credit_r):
        my = lax.axis_index("i")
        left = (my + N_DEV - 1) % N_DEV
        right = (my + 1) % N_DEV

        barrier = pltpu.get_barrier_semaphore()
        for nbr in (left, right):
            pl.semaphore_signal(barrier, inc=1, device_id=(nbr,),
                                device_id_type=pl.DeviceIdType.MESH)
        pl.semaphore_wait(barrier, 2)

        x8 = x_ref[...].astype(comm_dtype)
        w8 = w_ref[...].astype(comm_dtype)
        xbl[OWN, :, :] = x8
        wbl[OWN, :, :] = w8
        xbr[OWN, :, :] = x8
        wbr[OWN, :, :] = w8

        row0 = my * m_blk

        def accum(xb, wb, slot, first=False):
            tile = xb[slot, pl.ds(row0, m_blk), :].astype(jnp.bfloat16)
            part = jnp.dot(tile, wb[slot].astype(jnp.bfloat16),
                           preferred_element_type=jnp.float32)
            if first:
                out_ref[...] = part
            else:
                out_ref[...] += part

        def hop(xb, wb, sx, rx, sw, rw, dst, s, r):
            hx = pltpu.make_async_remote_copy(
                src_ref=xb.at[s], dst_ref=xb.at[r],
                send_sem=sx.at[s % 2], recv_sem=rx.at[r],
                device_id=(dst,), device_id_type=pl.DeviceIdType.MESH)
            hw = pltpu.make_async_remote_copy(
                src_ref=wb.at[s], dst_ref=wb.at[r],
                send_sem=sw.at[s % 2], recv_sem=rw.at[r],
                device_id=(dst,), device_id_type=pl.DeviceIdType.MESH)
            hx.start()
            hw.start()
            return hx, hw

        accum(xbl, wbl, OWN, first=True)

        for h in range(H_L):
            s = OWN if h == 0 else h % 2
            r = (h + 1) % 2
            if h >= 2:
                pl.semaphore_wait(credit_l, 1)
            if 2 <= h < H_R:
                pl.semaphore_wait(credit_r, 1)
            lx, lw = hop(xbl, wbl, sxl, rxl, swl, rwl, left, s, r)
            if h < H_R:
                rx_, rw_ = hop(xbr, wbr, sxr, rxr, swr, rwr, right, s, r)
            lx.wait()
            lw.wait()
            if h < H_R:
                rx_.wait()
                rw_.wait()
            if 1 <= h <= H_L - 2:
                pl.semaphore_signal(credit_l, inc=1, device_id=(right,),
                                    device_id_type=pl.DeviceIdType.MESH)
            if 1 <= h <= H_R - 2:
                pl.semaphore_signal(credit_r, inc=1, device_id=(left,),
                                    device_id_type=pl.DeviceIdType.MESH)
            accum(xbl, wbl, r)
            if h < H_R:
                accum(xbr, wbr, r)

        sc = sx_ref[0] * sw_ref[0]
        y = out_ref[...] * sc
        z = jnp.clip(y, -60.0, 60.0)
        out_ref[...] = y / (1.0 + jnp.exp(-z))

    return pl.pallas_call(
        body,
        out_shape=jax.ShapeDtypeStruct((m_blk, n), jnp.float32),
        in_specs=[
            pl.BlockSpec(memory_space=pltpu.VMEM),
            pl.BlockSpec(memory_space=pltpu.VMEM),
            pl.BlockSpec(memory_space=pltpu.SMEM),
            pl.BlockSpec(memory_space=pltpu.SMEM),
        ],
        out_specs=pl.BlockSpec(memory_space=pltpu.VMEM),
        scratch_shapes=[
            pltpu.VMEM((3, m, k_per), comm_dtype),
            pltpu.VMEM((3, k_blk, n), comm_dtype),
            pltpu.VMEM((3, m, k_per), comm_dtype),
            pltpu.VMEM((3, k_blk, n), comm_dtype),
            pltpu.SemaphoreType.DMA((2,)),
            pltpu.SemaphoreType.DMA((2,)),
            pltpu.SemaphoreType.DMA((2,)),
            pltpu.SemaphoreType.DMA((2,)),
            pltpu.SemaphoreType.DMA((2,)),
            pltpu.SemaphoreType.DMA((2,)),
            pltpu.SemaphoreType.DMA((2,)),
            pltpu.SemaphoreType.DMA((2,)),
            pltpu.SemaphoreType.REGULAR,
            pltpu.SemaphoreType.REGULAR,
        ],
        compiler_params=pltpu.CompilerParams(collective_id=0),
    )(x, w_mat, scale_x, scale_w)


# device time: 321090 ns/iter; 2.3260x vs baseline; 1.8442x over previous
import jax
import jax.numpy as jnp
from jax import lax
from jax.experimental import pallas as pl
from jax.experimental.pallas import tpu as pltpu

N_DEV = 32
H_L = N_DEV // 2
H_R = N_DEV - 1 - H_L
OWN = 2


def kernel(x, w_mat, scale_x, scale_w):
    m, k_per = x.shape
    k_blk, n = w_mat.shape
    m_blk = m // N_DEV

    comm_dtype = jnp.float8_e5m2

    def body(x_ref, w_ref, sx_ref, sw_ref, out_ref,
             xbl, wbl, xbr, wbr,
             sxl, rxl, swl, rwl, sxr, rxr, swr, rwr,
             credit_l, credit_r):
        my = lax.axis_index("i")

        z = my // 8
        rem = my % 8
        yy = rem // 2
        bb = rem % 2
        xx = jnp.where(yy % 2 == 0, bb, 1 - bb)
        p0 = z * 4 + jnp.where(z % 2 == 0, yy, 3 - yy)
        zz_ = 3 - z
        p1 = 16 + zz_ * 4 + jnp.where(zz_ % 2 == 0, yy, 3 - yy)
        ringpos = jnp.where(xx == 0, p0, p1)

        def ring_to_mesh(p):
            z0 = p // 4
            y0 = jnp.where(z0 % 2 == 0, p % 4, 3 - p % 4)
            i0 = z0 * 8 + y0 * 2 + jnp.where(y0 % 2 == 0, 0, 1)
            q = p - 16
            zz1 = q // 4
            z1 = 3 - zz1
            y1 = jnp.where(zz1 % 2 == 0, q % 4, 3 - q % 4)
            i1 = z1 * 8 + y1 * 2 + jnp.where(y1 % 2 == 0, 1, 0)
            return jnp.where(p < 16, i0, i1)

        left = ring_to_mesh((ringpos + N_DEV - 1) % N_DEV)
        right = ring_to_mesh((ringpos + 1) % N_DEV)

        barrier = pltpu.get_barrier_semaphore()
        for nbr in (left, right):
            pl.semaphore_signal(barrier, inc=1, device_id=(nbr,),
                                device_id_type=pl.DeviceIdType.MESH)
        pl.semaphore_wait(barrier, 2)

        x8 = x_ref[...].astype(comm_dtype)
        w8 = w_ref[...].astype(comm_dtype)
        xbl[OWN, :, :] = x8
        wbl[OWN, :, :] = w8
        xbr[OWN, :, :] = x8
        wbr[OWN, :, :] = w8

        row0 = my * m_blk

        def accum(xb, wb, slot, first=False):
            tile = xb[slot, pl.ds(row0, m_blk), :].astype(jnp.bfloat16)
            part = jnp.dot(tile, wb[slot].astype(jnp.bfloat16),
                           preferred_element_type=jnp.float32)
            if first:
                out_ref[...] = part
            else:
                out_ref[...] += part

        def hop(xb, wb, sx, rx, sw, rw, dst, s, r):
            hx = pltpu.make_async_remote_copy(
                src_ref=xb.at[s], dst_ref=xb.at[r],
                send_sem=sx.at[s % 2], recv_sem=rx.at[r],
                device_id=(dst,), device_id_type=pl.DeviceIdType.MESH)
            hw = pltpu.make_async_remote_copy(
                src_ref=wb.at[s], dst_ref=wb.at[r],
                send_sem=sw.at[s % 2], recv_sem=rw.at[r],
                device_id=(dst,), device_id_type=pl.DeviceIdType.MESH)
            hx.start()
            hw.start()
            return hx, hw

        accum(xbl, wbl, OWN, first=True)

        for h in range(H_L):
            s = OWN if h == 0 else h % 2
            r = (h + 1) % 2
            if h >= 2:
                pl.semaphore_wait(credit_l, 1)
            if 2 <= h < H_R:
                pl.semaphore_wait(credit_r, 1)
            lx, lw = hop(xbl, wbl, sxl, rxl, swl, rwl, left, s, r)
            if h < H_R:
                rx_, rw_ = hop(xbr, wbr, sxr, rxr, swr, rwr, right, s, r)
            if h >= 1:
                accum(xbl, wbl, s)
                if h - 1 < H_R:
                    accum(xbr, wbr, s)
            lx.wait()
            lw.wait()
            if h < H_R:
                rx_.wait()
                rw_.wait()
            if 1 <= h <= H_L - 2:
                pl.semaphore_signal(credit_l, inc=1, device_id=(right,),
                                    device_id_type=pl.DeviceIdType.MESH)
            if 1 <= h <= H_R - 2:
                pl.semaphore_signal(credit_r, inc=1, device_id=(left,),
                                    device_id_type=pl.DeviceIdType.MESH)

        accum(xbl, wbl, H_L % 2)

        sc = sx_ref[0] * sw_ref[0]
        y = out_ref[...] * sc
        z = jnp.clip(y, -60.0, 60.0)
        out_ref[...] = y / (1.0 + jnp.exp(-z))

    return pl.pallas_call(
        body,
        out_shape=jax.ShapeDtypeStruct((m_blk, n), jnp.float32),
        in_specs=[
            pl.BlockSpec(memory_space=pltpu.VMEM),
            pl.BlockSpec(memory_space=pltpu.VMEM),
            pl.BlockSpec(memory_space=pltpu.SMEM),
            pl.BlockSpec(memory_space=pltpu.SMEM),
        ],
        out_specs=pl.BlockSpec(memory_space=pltpu.VMEM),
        scratch_shapes=[
            pltpu.VMEM((3, m, k_per), comm_dtype),
            pltpu.VMEM((3, k_blk, n), comm_dtype),
            pltpu.VMEM((3, m, k_per), comm_dtype),
            pltpu.VMEM((3, k_blk, n), comm_dtype),
            pltpu.SemaphoreType.DMA((2,)),
            pltpu.SemaphoreType.DMA((2,)),
            pltpu.SemaphoreType.DMA((2,)),
            pltpu.SemaphoreType.DMA((2,)),
            pltpu.SemaphoreType.DMA((2,)),
            pltpu.SemaphoreType.DMA((2,)),
            pltpu.SemaphoreType.DMA((2,)),
            pltpu.SemaphoreType.DMA((2,)),
            pltpu.SemaphoreType.REGULAR,
            pltpu.SemaphoreType.REGULAR,
        ],
        compiler_params=pltpu.CompilerParams(collective_id=0),
    )(x, w_mat, scale_x, scale_w)
